# baseline (speedup 1.0000x reference)
import functools

import jax
import jax.numpy as jnp
from jax import lax
from jax.experimental import pallas as pl
from jax.experimental.pallas import tpu as pltpu
from jax.experimental.pallas import tpu_sc as plsc

_NC = 2
_NS = 16
_LANES = 16


def _sc_spmm(src0, dst0, vals0, src1, dst1, vals1, tsrc, tdst, tvals,
             tab, zrows, *, voc, hh, nnz, nchunk, ch):
    kg = ch // 128
    nf = nnz // ch
    mesh = plsc.VectorSubcoreMesh(core_axis_name="c", subcore_axis_name="s")

    @functools.partial(
        pl.kernel,
        mesh=mesh,
        compiler_params=pltpu.CompilerParams(use_tc_tiling_on_sc=False),
        out_type=jax.ShapeDtypeStruct((_NC, voc, hh), jnp.float32),
        scratch_types=[
            pltpu.VMEM((2, ch), jnp.int32),
            pltpu.VMEM((4, kg, 128), jnp.int32),
            pltpu.VMEM((2, ch), jnp.float32),
            pltpu.VMEM((2, ch, hh), jnp.float32),
            pltpu.VMEM((2, ch, hh), jnp.float32),
            pltpu.VMEM_SHARED((voc, hh), jnp.float32),
            pltpu.SemaphoreType.DMA,
            pltpu.SemaphoreType.DMA,
            pltpu.SemaphoreType.DMA,
            pltpu.SemaphoreType.DMA,
            pltpu.SemaphoreType.DMA,
            pltpu.SemaphoreType.DMA,
        ],
    )
    def k(s0_hbm, d0_hbm, v0_hbm, s1_hbm, d1_hbm, v1_hbm, ts_hbm, td_hbm,
          tv_hbm, tab_hbm, z_hbm, out_hbm,
          src_v, dst_v, vals_v, rows_g, rows_s, h_sh,
          isem0, isem1, gsem0, gsem1, ssem0, ssem1):
        isems = (isem0, isem1)
        gsems = (gsem0, gsem1)
        ssems = (ssem0, ssem1)
        c = lax.axis_index("c")
        s = lax.axis_index("s")
        rpt = voc // _NS
        pltpu.sync_copy(z_hbm, h_sh.at[pl.ds(s * rpt, rpt)])
        plsc.subcore_barrier()

        cbias = c * (2 * voc)

        def fire_idx(g, u):
            b = u % 2
            q = g * _NS + s

            @pl.when(q < nf)
            def _p0():
                off = q * ch
                pltpu.async_copy(s0_hbm.at[pl.ds(off, ch)],
                                 src_v.at[b], isems[b])
                pltpu.async_copy(v0_hbm.at[pl.ds(off, ch)], vals_v.at[b],
                                 isems[b])
                for j in range(kg):
                    pltpu.async_copy(d0_hbm.at[pl.ds(off + j * 128, 128)],
                                     dst_v.at[u % 4, j], isems[b])

            @pl.when((q >= nf) & (q < 2 * nf))
            def _p1():
                off = (q - nf) * ch
                pltpu.async_copy(s1_hbm.at[pl.ds(off, ch)],
                                 src_v.at[b], isems[b])
                pltpu.async_copy(v1_hbm.at[pl.ds(off, ch)], vals_v.at[b],
                                 isems[b])
                for j in range(kg):
                    pltpu.async_copy(d1_hbm.at[pl.ds(off + j * 128, 128)],
                                     dst_v.at[u % 4, j], isems[b])

            @pl.when(q >= 2 * nf)
            def _pt():
                qt = q - 2 * nf
                off = qt * ch
                pltpu.async_copy(ts_hbm.at[pl.ds(off, ch)], src_v.at[b],
                                 isems[b])
                pltpu.async_copy(tv_hbm.at[pl.ds(off, ch)], vals_v.at[b],
                                 isems[b])
                for j in range(kg):
                    pltpu.async_copy(td_hbm.at[qt, j], dst_v.at[u % 4, j],
                                     isems[b])

        def wait_idx(b):
            pltpu.make_async_copy(v0_hbm.at[pl.ds(0, ch)], src_v.at[b],
                                  isems[b]).wait()
            pltpu.make_async_copy(v0_hbm.at[pl.ds(0, ch)], vals_v.at[b],
                                  isems[b]).wait()
            for j in range(kg):
                pltpu.make_async_copy(v0_hbm.at[pl.ds(0, 128)],
                                      dst_v.at[0, j], isems[b]).wait()

        def bias(g, b):
            q = g * _NS + s
            add = cbias + jnp.where((q >= nf) & (q < 2 * nf), voc, 0)

            def body(t, acc):
                sl = pl.ds(t * _LANES, _LANES)
                src_v[b, sl] = src_v[b, sl] + add
                return acc

            lax.fori_loop(0, ch // _LANES, body, 0)

        def fire_gather(b):
            for j in range(kg):
                pltpu.async_copy(
                    tab_hbm.at[src_v.at[b, pl.ds(j * 128, 128)]],
                    rows_g.at[b, pl.ds(j * 128, 128)], gsems[b])

        def wait_gather(b):
            for j in range(kg):
                pltpu.make_async_copy(tab_hbm.at[pl.ds(0, 128)],
                                      rows_g.at[b, pl.ds(j * 128, 128)],
                                      gsems[b]).wait()

        def scale(b):
            def body(t, acc):
                val16 = vals_v[b, pl.ds(t * _LANES, _LANES)]
                for i in range(_LANES):
                    e = t * _LANES + i
                    vs = val16[i]
                    for q in range(hh // _LANES):
                        sl = pl.ds(q * _LANES, _LANES)
                        rows_s[b, e, sl] = rows_g[b, e, sl] * vs
                return acc

            lax.fori_loop(0, ch // _LANES, body, 0)

        def fire_scatter(u):
            b = u % 2
            for j in range(kg):
                pltpu.async_copy(rows_s.at[b, pl.ds(j * 128, 128)],
                                 h_sh.at[dst_v.at[u % 4, j]], ssems[b],
                                 add=True)

        def wait_scatter(b):
            for j in range(kg):
                pltpu.make_async_copy(rows_s.at[b, pl.ds(0, 128)],
                                      h_sh.at[pl.ds(0, 128)], ssems[b]).wait()

        def when(pred, fn):
            if pred is None:
                fn()
            else:
                pl.when(pred)(fn)

        fire_idx(0, 0)
        wait_idx(0)
        bias(0, 0)
        fire_gather(0)
        fire_idx(1, 1)

        n4 = nchunk // 4

        def outer(g4, carry):
            for u in range(4):
                b = u % 2
                nb = 1 - b
                g = g4 * 4 + u
                p_next = (g4 < n4 - 1) if u == 3 else None
                p_n2 = (g4 < n4 - 1) if u >= 2 else None
                p_pp = (g4 > 0) if u < 2 else None
                when(p_next, lambda: wait_idx(nb))
                when(p_next, lambda: bias(g + 1, nb))
                when(p_next, lambda: fire_gather(nb))
                wait_gather(b)
                when(p_pp, lambda: wait_scatter(b))
                scale(b)
                fire_scatter(u)
                when(p_n2, lambda: fire_idx(g + 2, u + 2))
            return carry

        lax.fori_loop(0, n4, outer, 0)
        wait_scatter(0)
        wait_scatter(1)
        plsc.subcore_barrier()
        pltpu.sync_copy(h_sh.at[pl.ds(s * rpt, rpt)],
                        out_hbm.at[c, pl.ds(s * rpt, rpt)])

    return k(src0, dst0, vals0, src1, dst1, vals1, tsrc, tdst, tvals,
             tab, zrows)


def _tc_split(adj0, adj1):
    nnz = adj0.shape[1]
    cb = 131072
    g = (nnz + cb - 1) // cb

    def body(a0, a1, d0, s0, d1, s1):
        d0[...] = a0[0, :]
        s0[...] = a0[1, :]
        d1[...] = a1[0, :]
        s1[...] = a1[1, :]

    return pl.pallas_call(
        body,
        grid=(g,),
        in_specs=[pl.BlockSpec((2, cb), lambda i: (0, i)),
                  pl.BlockSpec((2, cb), lambda i: (0, i))],
        out_specs=[pl.BlockSpec((cb,), lambda i: (i,))] * 4,
        out_shape=[jax.ShapeDtypeStruct((nnz,), jnp.int32)] * 4,
    )(adj0, adj1)


def _tc_fuse(X, h, fcw, fcb2):
    b, voc = X.shape
    hid = h.shape[1]
    out_dim = fcw.shape[0]
    kt = 512
    nk = voc // kt

    def body(x_ref, h_ref, w_ref, b_ref, o_ref, acc_ref):
        ki = pl.program_id(0)

        @pl.when(ki == 0)
        def _init():
            acc_ref[...] = jnp.zeros_like(acc_ref)

        acc_ref[...] = acc_ref[...] + jnp.dot(
            x_ref[...], h_ref[...], preferred_element_type=jnp.float32,
            precision=lax.Precision.HIGHEST)

        @pl.when(ki == nk - 1)
        def _fin():
            o_ref[...] = jnp.dot(
                acc_ref[...], w_ref[...].T, preferred_element_type=jnp.float32,
                precision=lax.Precision.HIGHEST) + b_ref[...]

    return pl.pallas_call(
        body,
        grid=(nk,),
        in_specs=[
            pl.BlockSpec((b, kt), lambda k: (0, k)),
            pl.BlockSpec((kt, hid), lambda k: (k, 0)),
            pl.BlockSpec((out_dim, hid), lambda k: (0, 0)),
            pl.BlockSpec((1, out_dim), lambda k: (0, 0)),
        ],
        out_specs=pl.BlockSpec((b, out_dim), lambda k: (0, 0)),
        out_shape=jax.ShapeDtypeStruct((b, out_dim), jnp.float32),
        scratch_shapes=[pltpu.VMEM((b, out_dim), jnp.float32)],
    )(X, h, fcw, fcb2)


def kernel(adj0_indices, adj0_values, adj1_indices, adj1_values,
           X_dv, W0, W1, fc_w, fc_b):
    voc, hid = W0.shape
    nnz = adj0_values.shape[0]
    hh = hid // _NC
    ch = 512
    nf = nnz // ch
    rem = nnz - nf * ch
    unit = _NS * 4
    total = ((2 * nf + (2 if rem else 0) + unit - 1) // unit) * unit
    ntail = total - 2 * nf
    tpad = ntail * ch - 2 * rem

    dst0, src0, dst1, src1 = _tc_split(adj0_indices, adj1_indices)
    zc = jnp.zeros(((ntail - 2) * ch,), jnp.int32)
    pw = (0, ch - rem)
    tsrc = jnp.concatenate([jnp.pad(src0[nf * ch:], pw),
                            jnp.pad(src1[nf * ch:] + voc, pw), zc])
    tdst = jnp.concatenate([jnp.pad(dst0[nf * ch:], pw),
                            jnp.pad(dst1[nf * ch:], pw),
                            zc]).reshape(-1, ch // 128, 128)
    tvals = jnp.concatenate([jnp.pad(adj0_values[nf * ch:], pw),
                             jnp.pad(adj1_values[nf * ch:], pw),
                             zc.astype(jnp.float32)])
    tab = jnp.concatenate([W0[:, :hh], W1[:, :hh], W0[:, hh:], W1[:, hh:]])
    zrows = jnp.zeros((voc // _NS, hh), jnp.float32)
    nchunk = total // _NS

    parts = _sc_spmm(src0, dst0, adj0_values, src1, dst1, adj1_values,
                     tsrc, tdst, tvals, tab, zrows,
                     voc=voc, hh=hh, nnz=nnz, nchunk=nchunk, ch=ch)
    h = jnp.concatenate([parts[0], parts[1]], axis=1)
    return _tc_fuse(X_dv, h, fc_w, fc_b.reshape(1, -1))
